# baseline (speedup 1.0000x reference)
import jax
import jax.numpy as jnp
from jax.experimental import pallas as pl
from jax.experimental.pallas import tpu as pltpu

BATCH = 64
HW = 32
D = 256

REP = 4


def _body(row_ref, col_ref, out_hbm, tile, sem):
    col = col_ref[...]
    row = row_ref[...]
    tile[:, :, :, 0:D] = jax.lax.broadcast_in_dim(col, (REP, HW, HW, D), (2, 3))
    tile[:, :, :, D : 2 * D] = jax.lax.broadcast_in_dim(row, (REP, HW, HW, D), (1, 3))
    copies = [
        pltpu.make_async_copy(tile, out_hbm.at[pl.ds(i * REP, REP)], sem)
        for i in range(BATCH // REP)
    ]
    for c in copies:
        c.start()
    for c in copies:
        c.wait()


def kernel(row_embeddings, column_embeddings):
    row = row_embeddings[:HW]
    col = column_embeddings[:HW]

    out4 = pl.pallas_call(
        _body,
        in_specs=[
            pl.BlockSpec((HW, D), lambda: (0, 0)),
            pl.BlockSpec((HW, D), lambda: (0, 0)),
        ],
        out_specs=pl.BlockSpec(memory_space=pl.ANY),
        out_shape=jax.ShapeDtypeStruct((BATCH, HW, HW, 2 * D), jnp.float32),
        scratch_shapes=[
            pltpu.VMEM((REP, HW, HW, 2 * D), jnp.float32),
            pltpu.SemaphoreType.DMA,
        ],
    )(row, col)
    return out4.reshape(BATCH, HW * HW, 2 * D)
